# baseline (speedup 1.0000x reference)
import jax
import jax.numpy as jnp
from jax.experimental import pallas as pl
from jax.experimental.pallas import tpu as pltpu

LANE = 128
SUB = 8
NSEM = 8


def _rup(v, m):
    return ((v + m - 1) // m) * m


def _choose_tile(B):
    if B <= LANE:
        return LANE, LANE
    TM = min(2048, max(LANE, _rup(B, 2 * LANE) // 2))
    return TM, _rup(B, TM)


def _dec_kernel(ids_ref, tab_ref,
                w1h, b1h, w2h, b2h, w3th, b3h,
                out_ref, c0_ref, c1_ref, x_ref,
                w1v, b1v, w2v, b2v, w3tv, b3v,
                semt, semw):
    TM = out_ref.shape[0]
    base = pl.program_id(0) * TM

    wpairs = ((w1h, w1v), (b1h, b1v), (w2h, w2v), (b2h, b2v),
              (w3th, w3tv), (b3h, b3v))
    for src, dst in wpairs:
        pltpu.make_async_copy(src, dst, semw).start()

    for mi in range(TM):
        a0 = pl.multiple_of((ids_ref[base + mi, 0] >> 3) << 3, SUB)
        a1 = pl.multiple_of((ids_ref[base + mi, 1] >> 3) << 3, SUB)
        pltpu.make_async_copy(
            tab_ref.at[pl.ds(a0, SUB), :], c0_ref.at[mi],
            semt.at[mi % NSEM]).start()
        pltpu.make_async_copy(
            tab_ref.at[pl.ds(a1, SUB), :], c1_ref.at[mi],
            semt.at[mi % NSEM]).start()
    nper = 2 * TM // NSEM
    for s in range(NSEM):
        pltpu.make_async_copy(c0_ref.at[pl.ds(0, nper)],
                              c0_ref.at[pl.ds(0, nper)], semt.at[s]).wait()
    for src, dst in wpairs:
        pltpu.make_async_copy(src, dst, semw).wait()

    for mi in range(TM):
        s0 = (SUB - (ids_ref[base + mi, 0] & 7)) & 7
        s1 = (SUB - (ids_ref[base + mi, 1] & 7)) & 7
        r0 = pltpu.roll(c0_ref[mi], s0, 0)
        r1 = pltpu.roll(c1_ref[mi], s1, 0)
        x_ref[mi:mi + 1, :] = (r0 + r1)[0:1, :]

    h1 = jnp.tanh(
        jnp.dot(x_ref[...].astype(jnp.bfloat16), w1v[...].astype(jnp.bfloat16),
                preferred_element_type=jnp.float32) + b1v[...])
    h2 = jnp.tanh(
        jnp.dot(h1.astype(jnp.bfloat16), w2v[...].astype(jnp.bfloat16),
                preferred_element_type=jnp.float32) + b2v[...])
    out = jax.lax.dot_general(
        h2.astype(jnp.bfloat16), w3tv[...].astype(jnp.bfloat16),
        dimension_numbers=(((1,), (1,)), ((), ())),
        preferred_element_type=jnp.float32)
    out_ref[...] = out + b3v[...]


def kernel(reprs, w1, b1, w2, b2, w3t, b3, x_id):
    NR, D = reprs.shape
    H = w2.shape[0]
    O = w3t.shape[0]
    B = x_id.shape[0]
    TM, B_pad = _choose_tile(B)

    ids = x_id.astype(jnp.int32)
    if B_pad != B:
        ids = jnp.zeros((B_pad, 2), jnp.int32).at[:B].set(ids)
    b3r = b3.reshape(1, O)

    out = pl.pallas_call(
        _dec_kernel,
        out_shape=jax.ShapeDtypeStruct((B_pad, O), jnp.float32),
        grid=(B_pad // TM,),
        in_specs=[pl.BlockSpec(memory_space=pltpu.SMEM)]
                 + [pl.BlockSpec(memory_space=pl.ANY)] * 7,
        out_specs=pl.BlockSpec((TM, O), lambda i: (i, 0)),
        scratch_shapes=[
            pltpu.VMEM((TM, SUB, D), jnp.float32),
            pltpu.VMEM((TM, SUB, D), jnp.float32),
            pltpu.VMEM((TM, D), jnp.float32),
            pltpu.VMEM((D, H), jnp.float32),
            pltpu.VMEM((1, H), jnp.float32),
            pltpu.VMEM((H, H), jnp.float32),
            pltpu.VMEM((1, H), jnp.float32),
            pltpu.VMEM((O, H), jnp.float32),
            pltpu.VMEM((1, O), jnp.float32),
            pltpu.SemaphoreType.DMA((NSEM,)),
            pltpu.SemaphoreType.DMA,
        ],
        compiler_params=pltpu.CompilerParams(
            dimension_semantics=("parallel",),
            disable_bounds_checks=True),
    )(ids, reprs, w1, b1, w2, b2, w3t, b3r)
    return out[:B]


# speedup vs baseline: 1.0930x; 1.0930x over previous
import jax
import jax.numpy as jnp
from jax.experimental import pallas as pl
from jax.experimental.pallas import tpu as pltpu

LANE = 128
SUB = 8
NSEM = 8


def _rup(v, m):
    return ((v + m - 1) // m) * m


def _choose_tile(B):
    if B <= LANE:
        return LANE, LANE
    TM = min(2048, max(LANE, _rup(B, 2 * LANE) // 2))
    return TM, _rup(B, TM)


def _dec_kernel(ids_ref, tab_ref,
                w1h, b1h, w2h, b2h, w3th, b3h,
                out_ref, c0_ref, c1_ref, x_ref,
                w1v, b1v, w2v, b2v, w3tv, b3v,
                semt, semw):
    TM = out_ref.shape[0]
    base = pl.program_id(0) * TM

    wpairs = ((w1h, w1v), (b1h, b1v), (w2h, w2v), (b2h, b2v),
              (w3th, w3tv), (b3h, b3v))
    for src, dst in wpairs:
        pltpu.make_async_copy(src, dst, semw).start()

    G = TM // NSEM
    for g in range(NSEM):
        for mi in range(g * G, (g + 1) * G):
            a0 = pl.multiple_of((ids_ref[base + mi, 0] >> 3) << 3, SUB)
            a1 = pl.multiple_of((ids_ref[base + mi, 1] >> 3) << 3, SUB)
            pltpu.make_async_copy(
                tab_ref.at[pl.ds(a0, SUB), :], c0_ref.at[mi],
                semt.at[g]).start()
            pltpu.make_async_copy(
                tab_ref.at[pl.ds(a1, SUB), :], c1_ref.at[mi],
                semt.at[g]).start()

    nper = 2 * G
    for g in range(NSEM):
        pltpu.make_async_copy(c0_ref.at[pl.ds(0, nper)],
                              c0_ref.at[pl.ds(0, nper)], semt.at[g]).wait()
        for mi in range(g * G, (g + 1) * G):
            s0 = (SUB - (ids_ref[base + mi, 0] & 7)) & 7
            s1 = (SUB - (ids_ref[base + mi, 1] & 7)) & 7
            r0 = pltpu.roll(c0_ref[mi], s0, 0)
            r1 = pltpu.roll(c1_ref[mi], s1, 0)
            x_ref[mi:mi + 1, :] = (r0 + r1)[0:1, :]
    for src, dst in wpairs:
        pltpu.make_async_copy(src, dst, semw).wait()

    h1 = jnp.tanh(
        jnp.dot(x_ref[...].astype(jnp.bfloat16), w1v[...].astype(jnp.bfloat16),
                preferred_element_type=jnp.float32) + b1v[...])
    h2 = jnp.tanh(
        jnp.dot(h1.astype(jnp.bfloat16), w2v[...].astype(jnp.bfloat16),
                preferred_element_type=jnp.float32) + b2v[...])
    out = jax.lax.dot_general(
        h2.astype(jnp.bfloat16), w3tv[...].astype(jnp.bfloat16),
        dimension_numbers=(((1,), (1,)), ((), ())),
        preferred_element_type=jnp.float32)
    out_ref[...] = out + b3v[...]


def kernel(reprs, w1, b1, w2, b2, w3t, b3, x_id):
    NR, D = reprs.shape
    H = w2.shape[0]
    O = w3t.shape[0]
    B = x_id.shape[0]
    TM, B_pad = _choose_tile(B)

    ids = x_id.astype(jnp.int32)
    if B_pad != B:
        ids = jnp.zeros((B_pad, 2), jnp.int32).at[:B].set(ids)
    b3r = b3.reshape(1, O)

    out = pl.pallas_call(
        _dec_kernel,
        out_shape=jax.ShapeDtypeStruct((B_pad, O), jnp.float32),
        grid=(B_pad // TM,),
        in_specs=[pl.BlockSpec(memory_space=pltpu.SMEM)]
                 + [pl.BlockSpec(memory_space=pl.ANY)] * 7,
        out_specs=pl.BlockSpec((TM, O), lambda i: (i, 0)),
        scratch_shapes=[
            pltpu.VMEM((TM, SUB, D), jnp.float32),
            pltpu.VMEM((TM, SUB, D), jnp.float32),
            pltpu.VMEM((TM, D), jnp.float32),
            pltpu.VMEM((D, H), jnp.float32),
            pltpu.VMEM((1, H), jnp.float32),
            pltpu.VMEM((H, H), jnp.float32),
            pltpu.VMEM((1, H), jnp.float32),
            pltpu.VMEM((O, H), jnp.float32),
            pltpu.VMEM((1, O), jnp.float32),
            pltpu.SemaphoreType.DMA((NSEM,)),
            pltpu.SemaphoreType.DMA,
        ],
        compiler_params=pltpu.CompilerParams(
            dimension_semantics=("parallel",),
            disable_bounds_checks=True),
    )(ids, reprs, w1, b1, w2, b2, w3t, b3r)
    return out[:B]
